# baseline (speedup 1.0000x reference)
import functools

import jax
import jax.numpy as jnp
from jax import lax
from jax.experimental import pallas as pl
from jax.experimental.pallas import tpu as pltpu
from jax.experimental.pallas import tpu_sc as plsc

_EPS = 1e-3


def _sc_gather_rows(table, idx):
    R, D = table.shape
    M = idx.shape[0]
    info = plsc.get_sparse_core_info()
    NC, NS = info.num_cores, info.num_subcores
    NW = NC * NS
    rows_pw = M // NW
    n_pairs = rows_pw // 256

    mesh = plsc.VectorSubcoreMesh(core_axis_name="c", subcore_axis_name="s")

    @functools.partial(
        pl.kernel, mesh=mesh,
        out_type=jax.ShapeDtypeStruct((M, D), jnp.float32),
        scratch_types=[
            pltpu.VMEM((rows_pw,), jnp.int32),
            pltpu.VMEM((128, D), jnp.float32),
            pltpu.VMEM((128, D), jnp.float32),
            pltpu.SemaphoreType.DMA,
            pltpu.SemaphoreType.DMA,
        ],
    )
    def k(table_hbm, idx_hbm, out_hbm, idx_v, buf0, buf1, sem0, sem1):
        wid = lax.axis_index("s") * NC + lax.axis_index("c")
        base = wid * rows_pw
        pltpu.sync_copy(idx_hbm.at[pl.ds(base, rows_pw)], idx_v)

        def body(g, _):
            o0 = g * 256
            o1 = o0 + 128
            cp0 = pltpu.async_copy(
                table_hbm.at[idx_v.at[pl.ds(o0, 128)]], buf0, sem0)
            cp1 = pltpu.async_copy(
                table_hbm.at[idx_v.at[pl.ds(o1, 128)]], buf1, sem1)
            cp0.wait()
            pltpu.sync_copy(buf0, out_hbm.at[pl.ds(base + o0, 128)])
            cp1.wait()
            pltpu.sync_copy(buf1, out_hbm.at[pl.ds(base + o1, 128)])
            return _

        lax.fori_loop(0, n_pairs, body, 0)

    return k(table, idx)


def _sc_gather_sum(table, idx, K):
    R, D = table.shape
    NN = idx.shape[0] // K
    info = plsc.get_sparse_core_info()
    NC, NS = info.num_cores, info.num_subcores
    NW = NC * NS
    nodes_pw = NN // NW
    rows_pw = nodes_pw * K
    npc = 128 // K
    n_quads = nodes_pw // (4 * npc)
    nvec = D // 16

    mesh = plsc.VectorSubcoreMesh(core_axis_name="c", subcore_axis_name="s")

    @functools.partial(
        pl.kernel, mesh=mesh,
        out_type=jax.ShapeDtypeStruct((NN, D), jnp.float32),
        scratch_types=[
            pltpu.VMEM((rows_pw,), jnp.int32),
            [pltpu.VMEM((128, D), jnp.float32) for _ in range(4)],
            [pltpu.VMEM((npc, D), jnp.float32) for _ in range(4)],
            [pltpu.SemaphoreType.DMA for _ in range(4)],
            [pltpu.SemaphoreType.DMA for _ in range(4)],
        ],
    )
    def k(table_hbm, idx_hbm, out_hbm, idx_v, bufs, obufs, gsems, wsems):
        wid = lax.axis_index("s") * NC + lax.axis_index("c")
        nb = wid * nodes_pw
        pltpu.sync_copy(idx_hbm.at[pl.ds(wid * rows_pw, rows_pw)], idx_v)

        def gather(c, i):
            return pltpu.async_copy(
                table_hbm.at[idx_v.at[pl.ds(c * 128, 128)]], bufs[i], gsems[i])

        def reduce_chunk(buf, obuf):
            def node_body(i, _):
                for d in range(nvec):
                    acc = buf[K * i, pl.ds(16 * d, 16)]
                    for kk in range(1, K):
                        acc = acc + buf[K * i + kk, pl.ds(16 * d, 16)]
                    obuf[i, pl.ds(16 * d, 16)] = acc
                return _
            lax.fori_loop(0, npc, node_body, 0)

        for i in range(4):
            gather(i, i)

        def body(s, carry):
            for i in range(4):
                c = s * 4 + i
                pltpu.make_async_copy(
                    table_hbm.at[idx_v.at[pl.ds(c * 128, 128)]],
                    bufs[i], gsems[i]).wait()
                @pl.when(s > 0)
                def _drain(i=i):
                    pltpu.make_async_copy(
                        obufs[i], out_hbm.at[pl.ds(nb, npc)], wsems[i]).wait()
                reduce_chunk(bufs[i], obufs[i])
                pltpu.async_copy(
                    obufs[i], out_hbm.at[pl.ds(nb + c * npc, npc)], wsems[i])
                @pl.when(s < n_quads - 1)
                def _refill(c=c, i=i):
                    gather(c + 4, i)
            return carry

        lax.fori_loop(0, n_quads, body, 0)
        for i in range(4):
            pltpu.make_async_copy(
                obufs[i], out_hbm.at[pl.ds(nb, npc)], wsems[i]).wait()

    return k(table, idx)


def _edge_body(D, ef_ref, vsr_ref, g_ref, We_ref, be_ref, ge_ref, bbe_ref,
               out_ref, agg_ref):
    j = pl.program_id(1)
    ef = ef_ref[0]
    acc = jnp.dot(ef.astype(jnp.bfloat16), We_ref[0:D],
                  preferred_element_type=jnp.float32)
    acc += jnp.dot(vsr_ref[0, 0].astype(jnp.bfloat16), We_ref[D:2 * D],
                   preferred_element_type=jnp.float32)
    acc += jnp.dot(vsr_ref[1, 0].astype(jnp.bfloat16), We_ref[2 * D:3 * D],
                   preferred_element_type=jnp.float32)
    acc += jnp.dot(g_ref[0].astype(jnp.bfloat16), We_ref[3 * D:4 * D],
                   preferred_element_type=jnp.float32)
    acc += be_ref[...]
    h = jnp.maximum(acc, 0.0) + ef
    m = jnp.mean(h, axis=-1, keepdims=True)
    v = jnp.mean(h * h, axis=-1, keepdims=True) - m * m
    y = (h - m) * lax.rsqrt(v + _EPS) * ge_ref[...] + bbe_ref[...]
    out_ref[0] = y

    @pl.when(j == 0)
    def _():
        agg_ref[...] = jnp.zeros_like(agg_ref)

    agg_ref[0] += jnp.sum(y, axis=0, keepdims=True)


def _node_body(D, vf_ref, ce_ref, vl_ref, g_ref, Wn_ref, bn_ref, gn_ref,
               bbn_ref, out_ref, agg_ref):
    j = pl.program_id(1)
    vf = vf_ref[0]
    vl = vl_ref[0]
    denom = jnp.where(vl == 0.0, 1.0, vl)
    ce = (ce_ref[0].astype(jnp.float32) / denom).astype(jnp.bfloat16)
    acc = jnp.dot(vf.astype(jnp.bfloat16), Wn_ref[0:D],
                  preferred_element_type=jnp.float32)
    acc += jnp.dot(ce, Wn_ref[D:2 * D], preferred_element_type=jnp.float32)
    acc += jnp.dot(g_ref[0].astype(jnp.bfloat16), Wn_ref[2 * D:3 * D],
                   preferred_element_type=jnp.float32)
    acc += bn_ref[...]
    h = jnp.maximum(acc, 0.0) + vf
    m = jnp.mean(h, axis=-1, keepdims=True)
    v = jnp.mean(h * h, axis=-1, keepdims=True) - m * m
    y = (h - m) * lax.rsqrt(v + _EPS) * gn_ref[...] + bbn_ref[...]
    out_ref[0] = y

    @pl.when(j == 0)
    def _():
        agg_ref[...] = jnp.zeros_like(agg_ref)

    agg_ref[0] += jnp.sum(y, axis=0, keepdims=True)


def _global_body(D, g_ref, aggv_ref, agge_ref, vn_ref, ve_ref, Wg_ref,
                 bg_ref, gg_ref, bbg_ref, out_ref):
    g = g_ref[...]
    vn = vn_ref[...]
    ve = ve_ref[...]
    aggv = aggv_ref[...] / jnp.where(vn == 0.0, 1.0, vn)
    agge = agge_ref[...] / jnp.where(ve == 0.0, 1.0, ve)
    acc = jnp.dot(g, Wg_ref[0:D], preferred_element_type=jnp.float32)
    acc += jnp.dot(aggv, Wg_ref[D:2 * D], preferred_element_type=jnp.float32)
    acc += jnp.dot(agge, Wg_ref[2 * D:3 * D], preferred_element_type=jnp.float32)
    acc += bg_ref[...]
    h = jnp.maximum(acc, 0.0) + g
    out_ref[...] = h * (1.0 / jnp.sqrt(1.0 + _EPS)) * gg_ref[...] + bbg_ref[...]


def kernel(vertex_feat, edges_feat, global_feat, edges_idx,
           connected_edges_idx, valid_lens, valid_nodes, valid_edges,
           W_e, b_e, g_e, beta_e, W_n, b_n, g_n, beta_n, W_g, b_g, g_g, beta_g):
    B, N, D = vertex_feat.shape
    E = edges_feat.shape[1]
    K = connected_edges_idx.shape[2]
    BE = 1024
    BN = 1024
    NH = 4
    B2 = B // NH

    gf3 = global_feat[:, None, :]
    b_e2 = b_e.reshape(1, D)
    g_e2 = g_e.reshape(1, D)
    beta_e2 = beta_e.reshape(1, D)
    b_n2 = b_n.reshape(1, D)
    g_n2 = g_n.reshape(1, D)
    beta_n2 = beta_n.reshape(1, D)
    We_bf = W_e.astype(jnp.bfloat16)
    Wn_bf = W_n.astype(jnp.bfloat16)
    vl3 = valid_lens.astype(jnp.float32)[:, :, None]
    offs_n2 = (jnp.arange(B2, dtype=jnp.int32) * N)[:, None]
    offs_e2 = (jnp.arange(B2, dtype=jnp.int32) * E)[:, None, None]

    vtab = vertex_feat.reshape(B * N, D)

    def edge_stage(h):
        ei = edges_idx[h * B2:(h + 1) * B2]
        src = (ei[:, :, 0] + offs_n2 + h * B2 * N).reshape(-1)
        dst = (ei[:, :, 1] + offs_n2 + h * B2 * N).reshape(-1)
        vsr = _sc_gather_rows(vtab, jnp.concatenate([src, dst]))
        vsr = vsr.reshape(2, B2, E, D)
        return pl.pallas_call(
            functools.partial(_edge_body, D),
            grid=(B2, E // BE),
            in_specs=[
                pl.BlockSpec((1, BE, D), lambda b, j: (b + h * B2, j, 0)),
                pl.BlockSpec((2, 1, BE, D), lambda b, j: (0, b, j, 0)),
                pl.BlockSpec((1, 1, D), lambda b, j: (b + h * B2, 0, 0)),
                pl.BlockSpec((4 * D, D), lambda b, j: (0, 0)),
                pl.BlockSpec((1, D), lambda b, j: (0, 0)),
                pl.BlockSpec((1, D), lambda b, j: (0, 0)),
                pl.BlockSpec((1, D), lambda b, j: (0, 0)),
            ],
            out_specs=[
                pl.BlockSpec((1, BE, D), lambda b, j: (b, j, 0)),
                pl.BlockSpec((1, 1, D), lambda b, j: (b, 0, 0)),
            ],
            out_shape=[
                jax.ShapeDtypeStruct((B2, E, D), jnp.float32),
                jax.ShapeDtypeStruct((B2, 1, D), jnp.float32),
            ],
        )(edges_feat, vsr, gf3, We_bf, b_e2, g_e2, beta_e2)

    def node_stage(h, edges_new_h):
        cidx = (connected_edges_idx[h * B2:(h + 1) * B2] + offs_e2).reshape(-1)
        ce = _sc_gather_sum(edges_new_h.reshape(B2 * E, D), cidx, K)
        ce = ce.reshape(B2, N, D)
        return pl.pallas_call(
            functools.partial(_node_body, D),
            grid=(B2, N // BN),
            in_specs=[
                pl.BlockSpec((1, BN, D), lambda b, j: (b + h * B2, j, 0)),
                pl.BlockSpec((1, BN, D), lambda b, j: (b, j, 0)),
                pl.BlockSpec((1, BN, 1), lambda b, j: (b + h * B2, j, 0)),
                pl.BlockSpec((1, 1, D), lambda b, j: (b + h * B2, 0, 0)),
                pl.BlockSpec((3 * D, D), lambda b, j: (0, 0)),
                pl.BlockSpec((1, D), lambda b, j: (0, 0)),
                pl.BlockSpec((1, D), lambda b, j: (0, 0)),
                pl.BlockSpec((1, D), lambda b, j: (0, 0)),
            ],
            out_specs=[
                pl.BlockSpec((1, BN, D), lambda b, j: (b, j, 0)),
                pl.BlockSpec((1, 1, D), lambda b, j: (b, 0, 0)),
            ],
            out_shape=[
                jax.ShapeDtypeStruct((B2, N, D), jnp.float32),
                jax.ShapeDtypeStruct((B2, 1, D), jnp.float32),
            ],
        )(vertex_feat, ce, vl3, gf3, Wn_bf, b_n2, g_n2, beta_n2)

    en_h, ae_h, vn_h, av_h = [], [], [], []
    for h in range(NH):
        en, ae = edge_stage(h)
        en_h.append(en)
        ae_h.append(ae)
    for h in range(NH):
        vn, av = node_stage(h, en_h[h])
        vn_h.append(vn)
        av_h.append(av)
    edges_new = jnp.concatenate(en_h, axis=0)
    vertex_new = jnp.concatenate(vn_h, axis=0)
    agg_e = jnp.concatenate(ae_h, axis=0)
    agg_v = jnp.concatenate(av_h, axis=0)

    vn2 = valid_nodes.astype(jnp.float32)[:, None]
    ve2 = valid_edges.astype(jnp.float32)[:, None]
    b_g2 = b_g.reshape(1, D)
    g_g2 = g_g.reshape(1, D)
    beta_g2 = beta_g.reshape(1, D)
    global_new = pl.pallas_call(
        functools.partial(_global_body, D),
        out_shape=jax.ShapeDtypeStruct((B, D), jnp.float32),
    )(global_feat, agg_v.reshape(B, D), agg_e.reshape(B, D), vn2, ve2,
      W_g, b_g2, g_g2, beta_g2)

    return (vertex_new, edges_new, global_new)


# speedup vs baseline: 1.0046x; 1.0046x over previous
import functools

import jax
import jax.numpy as jnp
from jax import lax
from jax.experimental import pallas as pl
from jax.experimental.pallas import tpu as pltpu
from jax.experimental.pallas import tpu_sc as plsc

_EPS = 1e-3


def _sc_gather_rows(table, idx):
    R, D = table.shape
    M = idx.shape[0]
    info = plsc.get_sparse_core_info()
    NC, NS = info.num_cores, info.num_subcores
    NW = NC * NS
    rows_pw = M // NW
    n_pairs = rows_pw // 256

    mesh = plsc.VectorSubcoreMesh(core_axis_name="c", subcore_axis_name="s")

    @functools.partial(
        pl.kernel, mesh=mesh,
        out_type=jax.ShapeDtypeStruct((M, D), jnp.float32),
        scratch_types=[
            pltpu.VMEM((rows_pw,), jnp.int32),
            pltpu.VMEM((128, D), jnp.float32),
            pltpu.VMEM((128, D), jnp.float32),
            pltpu.SemaphoreType.DMA,
            pltpu.SemaphoreType.DMA,
        ],
    )
    def k(table_hbm, idx_hbm, out_hbm, idx_v, buf0, buf1, sem0, sem1):
        wid = lax.axis_index("s") * NC + lax.axis_index("c")
        base = wid * rows_pw
        pltpu.sync_copy(idx_hbm.at[pl.ds(base, rows_pw)], idx_v)

        def body(g, _):
            o0 = g * 256
            o1 = o0 + 128
            cp0 = pltpu.async_copy(
                table_hbm.at[idx_v.at[pl.ds(o0, 128)]], buf0, sem0)
            cp1 = pltpu.async_copy(
                table_hbm.at[idx_v.at[pl.ds(o1, 128)]], buf1, sem1)
            cp0.wait()
            pltpu.sync_copy(buf0, out_hbm.at[pl.ds(base + o0, 128)])
            cp1.wait()
            pltpu.sync_copy(buf1, out_hbm.at[pl.ds(base + o1, 128)])
            return _

        lax.fori_loop(0, n_pairs, body, 0)

    return k(table, idx)


def _sc_gather_sum(table, idx, K):
    R, D = table.shape
    NN = idx.shape[0] // K
    info = plsc.get_sparse_core_info()
    NC, NS = info.num_cores, info.num_subcores
    NW = NC * NS
    nodes_pw = NN // NW
    rows_pw = nodes_pw * K
    npc = 128 // K
    n_quads = nodes_pw // (4 * npc)
    nvec = D // 16

    mesh = plsc.VectorSubcoreMesh(core_axis_name="c", subcore_axis_name="s")

    @functools.partial(
        pl.kernel, mesh=mesh,
        out_type=jax.ShapeDtypeStruct((NN, D), jnp.float32),
        scratch_types=[
            pltpu.VMEM((rows_pw,), jnp.int32),
            [pltpu.VMEM((128, D), jnp.float32) for _ in range(4)],
            [pltpu.VMEM((npc, D), jnp.float32) for _ in range(4)],
            [pltpu.SemaphoreType.DMA for _ in range(4)],
            [pltpu.SemaphoreType.DMA for _ in range(4)],
        ],
    )
    def k(table_hbm, idx_hbm, out_hbm, idx_v, bufs, obufs, gsems, wsems):
        wid = lax.axis_index("s") * NC + lax.axis_index("c")
        nb = wid * nodes_pw
        pltpu.sync_copy(idx_hbm.at[pl.ds(wid * rows_pw, rows_pw)], idx_v)

        def gather(c, i):
            return pltpu.async_copy(
                table_hbm.at[idx_v.at[pl.ds(c * 128, 128)]], bufs[i], gsems[i])

        def reduce_chunk(buf, obuf):
            def node_body(i, _):
                for d in range(nvec):
                    acc = buf[K * i, pl.ds(16 * d, 16)]
                    for kk in range(1, K):
                        acc = acc + buf[K * i + kk, pl.ds(16 * d, 16)]
                    obuf[i, pl.ds(16 * d, 16)] = acc
                return _
            lax.fori_loop(0, npc, node_body, 0)

        for i in range(4):
            gather(i, i)

        def body(s, carry):
            for i in range(4):
                c = s * 4 + i
                pltpu.make_async_copy(
                    table_hbm.at[idx_v.at[pl.ds(c * 128, 128)]],
                    bufs[i], gsems[i]).wait()
                @pl.when(s > 0)
                def _drain(i=i):
                    pltpu.make_async_copy(
                        obufs[i], out_hbm.at[pl.ds(nb, npc)], wsems[i]).wait()
                reduce_chunk(bufs[i], obufs[i])
                pltpu.async_copy(
                    obufs[i], out_hbm.at[pl.ds(nb + c * npc, npc)], wsems[i])
                @pl.when(s < n_quads - 1)
                def _refill(c=c, i=i):
                    gather(c + 4, i)
            return carry

        lax.fori_loop(0, n_quads, body, 0)
        for i in range(4):
            pltpu.make_async_copy(
                obufs[i], out_hbm.at[pl.ds(nb, npc)], wsems[i]).wait()

    return k(table, idx)


def _edge_body(D, ef_ref, vsr_ref, g_ref, We_ref, be_ref, ge_ref, bbe_ref,
               out_ref, agg_ref):
    j = pl.program_id(1)
    ef = ef_ref[0]
    acc = jnp.dot(ef.astype(jnp.bfloat16), We_ref[0:D],
                  preferred_element_type=jnp.float32)
    acc += jnp.dot(vsr_ref[0, 0].astype(jnp.bfloat16), We_ref[D:2 * D],
                   preferred_element_type=jnp.float32)
    acc += jnp.dot(vsr_ref[1, 0].astype(jnp.bfloat16), We_ref[2 * D:3 * D],
                   preferred_element_type=jnp.float32)
    acc += jnp.dot(g_ref[0].astype(jnp.bfloat16), We_ref[3 * D:4 * D],
                   preferred_element_type=jnp.float32)
    acc += be_ref[...]
    h = jnp.maximum(acc, 0.0) + ef
    m = jnp.mean(h, axis=-1, keepdims=True)
    v = jnp.mean(h * h, axis=-1, keepdims=True) - m * m
    y = (h - m) * lax.rsqrt(v + _EPS) * ge_ref[...] + bbe_ref[...]
    out_ref[0] = y

    @pl.when(j == 0)
    def _():
        agg_ref[...] = jnp.zeros_like(agg_ref)

    agg_ref[0] += jnp.sum(y, axis=0, keepdims=True)


def _node_body(D, vf_ref, ce_ref, vl_ref, g_ref, Wn_ref, bn_ref, gn_ref,
               bbn_ref, out_ref, agg_ref):
    j = pl.program_id(1)
    vf = vf_ref[0]
    vl = vl_ref[0]
    denom = jnp.where(vl == 0.0, 1.0, vl)
    ce = (ce_ref[0].astype(jnp.float32) / denom).astype(jnp.bfloat16)
    acc = jnp.dot(vf.astype(jnp.bfloat16), Wn_ref[0:D],
                  preferred_element_type=jnp.float32)
    acc += jnp.dot(ce, Wn_ref[D:2 * D], preferred_element_type=jnp.float32)
    acc += jnp.dot(g_ref[0].astype(jnp.bfloat16), Wn_ref[2 * D:3 * D],
                   preferred_element_type=jnp.float32)
    acc += bn_ref[...]
    h = jnp.maximum(acc, 0.0) + vf
    m = jnp.mean(h, axis=-1, keepdims=True)
    v = jnp.mean(h * h, axis=-1, keepdims=True) - m * m
    y = (h - m) * lax.rsqrt(v + _EPS) * gn_ref[...] + bbn_ref[...]
    out_ref[0] = y

    @pl.when(j == 0)
    def _():
        agg_ref[...] = jnp.zeros_like(agg_ref)

    agg_ref[0] += jnp.sum(y, axis=0, keepdims=True)


def _global_body(D, g_ref, aggv_ref, agge_ref, vn_ref, ve_ref, Wg_ref,
                 bg_ref, gg_ref, bbg_ref, out_ref):
    g = g_ref[...]
    vn = vn_ref[...]
    ve = ve_ref[...]
    aggv = aggv_ref[...] / jnp.where(vn == 0.0, 1.0, vn)
    agge = agge_ref[...] / jnp.where(ve == 0.0, 1.0, ve)
    acc = jnp.dot(g, Wg_ref[0:D], preferred_element_type=jnp.float32)
    acc += jnp.dot(aggv, Wg_ref[D:2 * D], preferred_element_type=jnp.float32)
    acc += jnp.dot(agge, Wg_ref[2 * D:3 * D], preferred_element_type=jnp.float32)
    acc += bg_ref[...]
    h = jnp.maximum(acc, 0.0) + g
    out_ref[...] = h * (1.0 / jnp.sqrt(1.0 + _EPS)) * gg_ref[...] + bbg_ref[...]


def kernel(vertex_feat, edges_feat, global_feat, edges_idx,
           connected_edges_idx, valid_lens, valid_nodes, valid_edges,
           W_e, b_e, g_e, beta_e, W_n, b_n, g_n, beta_n, W_g, b_g, g_g, beta_g):
    B, N, D = vertex_feat.shape
    E = edges_feat.shape[1]
    K = connected_edges_idx.shape[2]
    BE = 1024
    BN = 1024
    NH = 2
    B2 = B // NH

    gf3 = global_feat[:, None, :]
    b_e2 = b_e.reshape(1, D)
    g_e2 = g_e.reshape(1, D)
    beta_e2 = beta_e.reshape(1, D)
    b_n2 = b_n.reshape(1, D)
    g_n2 = g_n.reshape(1, D)
    beta_n2 = beta_n.reshape(1, D)
    We_bf = W_e.astype(jnp.bfloat16)
    Wn_bf = W_n.astype(jnp.bfloat16)
    vl3 = valid_lens.astype(jnp.float32)[:, :, None]
    offs_n2 = (jnp.arange(B2, dtype=jnp.int32) * N)[:, None]
    offs_e2 = (jnp.arange(B2, dtype=jnp.int32) * E)[:, None, None]

    vtab = vertex_feat.reshape(B * N, D)

    def edge_stage(h):
        ei = edges_idx[h * B2:(h + 1) * B2]
        src = (ei[:, :, 0] + offs_n2 + h * B2 * N).reshape(-1)
        dst = (ei[:, :, 1] + offs_n2 + h * B2 * N).reshape(-1)
        vsr = _sc_gather_rows(vtab, jnp.concatenate([src, dst]))
        vsr = vsr.reshape(2, B2, E, D)
        return pl.pallas_call(
            functools.partial(_edge_body, D),
            grid=(B2, E // BE),
            in_specs=[
                pl.BlockSpec((1, BE, D), lambda b, j: (b + h * B2, j, 0)),
                pl.BlockSpec((2, 1, BE, D), lambda b, j: (0, b, j, 0)),
                pl.BlockSpec((1, 1, D), lambda b, j: (b + h * B2, 0, 0)),
                pl.BlockSpec((4 * D, D), lambda b, j: (0, 0)),
                pl.BlockSpec((1, D), lambda b, j: (0, 0)),
                pl.BlockSpec((1, D), lambda b, j: (0, 0)),
                pl.BlockSpec((1, D), lambda b, j: (0, 0)),
            ],
            out_specs=[
                pl.BlockSpec((1, BE, D), lambda b, j: (b, j, 0)),
                pl.BlockSpec((1, 1, D), lambda b, j: (b, 0, 0)),
            ],
            out_shape=[
                jax.ShapeDtypeStruct((B2, E, D), jnp.float32),
                jax.ShapeDtypeStruct((B2, 1, D), jnp.float32),
            ],
        )(edges_feat, vsr, gf3, We_bf, b_e2, g_e2, beta_e2)

    def node_stage(h, edges_new_h):
        cidx = (connected_edges_idx[h * B2:(h + 1) * B2] + offs_e2).reshape(-1)
        ce = _sc_gather_sum(edges_new_h.reshape(B2 * E, D), cidx, K)
        ce = ce.reshape(B2, N, D)
        return pl.pallas_call(
            functools.partial(_node_body, D),
            grid=(B2, N // BN),
            in_specs=[
                pl.BlockSpec((1, BN, D), lambda b, j: (b + h * B2, j, 0)),
                pl.BlockSpec((1, BN, D), lambda b, j: (b, j, 0)),
                pl.BlockSpec((1, BN, 1), lambda b, j: (b + h * B2, j, 0)),
                pl.BlockSpec((1, 1, D), lambda b, j: (b + h * B2, 0, 0)),
                pl.BlockSpec((3 * D, D), lambda b, j: (0, 0)),
                pl.BlockSpec((1, D), lambda b, j: (0, 0)),
                pl.BlockSpec((1, D), lambda b, j: (0, 0)),
                pl.BlockSpec((1, D), lambda b, j: (0, 0)),
            ],
            out_specs=[
                pl.BlockSpec((1, BN, D), lambda b, j: (b, j, 0)),
                pl.BlockSpec((1, 1, D), lambda b, j: (b, 0, 0)),
            ],
            out_shape=[
                jax.ShapeDtypeStruct((B2, N, D), jnp.float32),
                jax.ShapeDtypeStruct((B2, 1, D), jnp.float32),
            ],
        )(vertex_feat, ce, vl3, gf3, Wn_bf, b_n2, g_n2, beta_n2)

    en_h, ae_h, vn_h, av_h = [], [], [], []
    for h in range(NH):
        en, ae = edge_stage(h)
        en_h.append(en)
        ae_h.append(ae)
    for h in range(NH):
        vn, av = node_stage(h, en_h[h])
        vn_h.append(vn)
        av_h.append(av)
    edges_new = jnp.concatenate(en_h, axis=0)
    vertex_new = jnp.concatenate(vn_h, axis=0)
    agg_e = jnp.concatenate(ae_h, axis=0)
    agg_v = jnp.concatenate(av_h, axis=0)

    vn2 = valid_nodes.astype(jnp.float32)[:, None]
    ve2 = valid_edges.astype(jnp.float32)[:, None]
    b_g2 = b_g.reshape(1, D)
    g_g2 = g_g.reshape(1, D)
    beta_g2 = beta_g.reshape(1, D)
    global_new = pl.pallas_call(
        functools.partial(_global_body, D),
        out_shape=jax.ShapeDtypeStruct((B, D), jnp.float32),
    )(global_feat, agg_v.reshape(B, D), agg_e.reshape(B, D), vn2, ve2,
      W_g, b_g2, g_g2, beta_g2)

    return (vertex_new, edges_new, global_new)


# speedup vs baseline: 1.0629x; 1.0581x over previous
import functools

import jax
import jax.numpy as jnp
from jax import lax
from jax.experimental import pallas as pl
from jax.experimental.pallas import tpu as pltpu
from jax.experimental.pallas import tpu_sc as plsc

_EPS = 1e-3


def _sc_gather_rows(table, idx):
    R, D = table.shape
    M = idx.shape[0]
    info = plsc.get_sparse_core_info()
    NC, NS = info.num_cores, info.num_subcores
    NW = NC * NS
    rows_pw = M // NW
    n_pairs = rows_pw // 256

    mesh = plsc.VectorSubcoreMesh(core_axis_name="c", subcore_axis_name="s")

    @functools.partial(
        pl.kernel, mesh=mesh,
        out_type=jax.ShapeDtypeStruct((M, D), jnp.float32),
        scratch_types=[
            pltpu.VMEM((rows_pw,), jnp.int32),
            pltpu.VMEM((128, D), jnp.float32),
            pltpu.VMEM((128, D), jnp.float32),
            pltpu.SemaphoreType.DMA,
            pltpu.SemaphoreType.DMA,
        ],
    )
    def k(table_hbm, idx_hbm, out_hbm, idx_v, buf0, buf1, sem0, sem1):
        wid = lax.axis_index("s") * NC + lax.axis_index("c")
        base = wid * rows_pw
        pltpu.sync_copy(idx_hbm.at[pl.ds(base, rows_pw)], idx_v)

        def body(g, _):
            o0 = g * 256
            o1 = o0 + 128
            cp0 = pltpu.async_copy(
                table_hbm.at[idx_v.at[pl.ds(o0, 128)]], buf0, sem0)
            cp1 = pltpu.async_copy(
                table_hbm.at[idx_v.at[pl.ds(o1, 128)]], buf1, sem1)
            cp0.wait()
            pltpu.sync_copy(buf0, out_hbm.at[pl.ds(base + o0, 128)])
            cp1.wait()
            pltpu.sync_copy(buf1, out_hbm.at[pl.ds(base + o1, 128)])
            return _

        lax.fori_loop(0, n_pairs, body, 0)

    return k(table, idx)


def _sc_gather_sum(table, idx, K):
    R, D = table.shape
    NN = idx.shape[0] // K
    info = plsc.get_sparse_core_info()
    NC, NS = info.num_cores, info.num_subcores
    NW = NC * NS
    nodes_pw = NN // NW
    rows_pw = nodes_pw * K
    npc = 128 // K
    n_quads = nodes_pw // (4 * npc)
    nvec = D // 16

    mesh = plsc.VectorSubcoreMesh(core_axis_name="c", subcore_axis_name="s")

    @functools.partial(
        pl.kernel, mesh=mesh,
        out_type=jax.ShapeDtypeStruct((NN, D), jnp.float32),
        scratch_types=[
            pltpu.VMEM((rows_pw,), jnp.int32),
            [pltpu.VMEM((128, D), jnp.float32) for _ in range(4)],
            [pltpu.VMEM((npc, D), jnp.float32) for _ in range(4)],
            [pltpu.SemaphoreType.DMA for _ in range(4)],
            [pltpu.SemaphoreType.DMA for _ in range(4)],
        ],
    )
    def k(table_hbm, idx_hbm, out_hbm, idx_v, bufs, obufs, gsems, wsems):
        wid = lax.axis_index("s") * NC + lax.axis_index("c")
        nb = wid * nodes_pw
        pltpu.sync_copy(idx_hbm.at[pl.ds(wid * rows_pw, rows_pw)], idx_v)

        def gather(c, i):
            return pltpu.async_copy(
                table_hbm.at[idx_v.at[pl.ds(c * 128, 128)]], bufs[i], gsems[i])

        def reduce_chunk(buf, obuf):
            def node_body(i, _):
                for d in range(nvec):
                    acc = buf[K * i, pl.ds(16 * d, 16)]
                    for kk in range(1, K):
                        acc = acc + buf[K * i + kk, pl.ds(16 * d, 16)]
                    obuf[i, pl.ds(16 * d, 16)] = acc
                return _
            lax.fori_loop(0, npc, node_body, 0)

        for i in range(4):
            gather(i, i)

        def body(s, carry):
            for i in range(4):
                c = s * 4 + i
                pltpu.make_async_copy(
                    table_hbm.at[idx_v.at[pl.ds(c * 128, 128)]],
                    bufs[i], gsems[i]).wait()
                @pl.when(s > 0)
                def _drain(i=i):
                    pltpu.make_async_copy(
                        obufs[i], out_hbm.at[pl.ds(nb, npc)], wsems[i]).wait()
                reduce_chunk(bufs[i], obufs[i])
                pltpu.async_copy(
                    obufs[i], out_hbm.at[pl.ds(nb + c * npc, npc)], wsems[i])
                @pl.when(s < n_quads - 1)
                def _refill(c=c, i=i):
                    gather(c + 4, i)
            return carry

        lax.fori_loop(0, n_quads, body, 0)
        for i in range(4):
            pltpu.make_async_copy(
                obufs[i], out_hbm.at[pl.ds(nb, npc)], wsems[i]).wait()

    return k(table, idx)


def _edge_body(D, ef_ref, vsr_ref, g_ref, We_ref, be_ref, ge_ref, bbe_ref,
               out_ref, agg_ref):
    j = pl.program_id(1)
    ef = ef_ref[0]
    acc = jnp.dot(ef.astype(jnp.bfloat16), We_ref[0:D],
                  preferred_element_type=jnp.float32)
    acc += jnp.dot(vsr_ref[0, 0].astype(jnp.bfloat16), We_ref[D:2 * D],
                   preferred_element_type=jnp.float32)
    acc += jnp.dot(vsr_ref[1, 0].astype(jnp.bfloat16), We_ref[2 * D:3 * D],
                   preferred_element_type=jnp.float32)
    acc += jnp.dot(g_ref[0].astype(jnp.bfloat16), We_ref[3 * D:4 * D],
                   preferred_element_type=jnp.float32)
    acc += be_ref[...]
    h = jnp.maximum(acc, 0.0) + ef
    m = jnp.mean(h, axis=-1, keepdims=True)
    v = jnp.mean(h * h, axis=-1, keepdims=True) - m * m
    y = (h - m) * lax.rsqrt(v + _EPS) * ge_ref[...] + bbe_ref[...]
    out_ref[0] = y

    @pl.when(j == 0)
    def _():
        agg_ref[...] = jnp.zeros_like(agg_ref)

    agg_ref[0] += jnp.sum(y, axis=0, keepdims=True)


def _node_body(D, vf_ref, ce_ref, vl_ref, g_ref, Wn_ref, bn_ref, gn_ref,
               bbn_ref, *refs):
    if len(refs) == 3:
        prev_ref, out_ref, agg_ref = refs
        del prev_ref
    else:
        out_ref, agg_ref = refs
    j = pl.program_id(1)
    vf = vf_ref[0]
    vl = vl_ref[0]
    denom = jnp.where(vl == 0.0, 1.0, vl)
    ce = (ce_ref[0].astype(jnp.float32) / denom).astype(jnp.bfloat16)
    acc = jnp.dot(vf.astype(jnp.bfloat16), Wn_ref[0:D],
                  preferred_element_type=jnp.float32)
    acc += jnp.dot(ce, Wn_ref[D:2 * D], preferred_element_type=jnp.float32)
    acc += jnp.dot(g_ref[0].astype(jnp.bfloat16), Wn_ref[2 * D:3 * D],
                   preferred_element_type=jnp.float32)
    acc += bn_ref[...]
    h = jnp.maximum(acc, 0.0) + vf
    m = jnp.mean(h, axis=-1, keepdims=True)
    v = jnp.mean(h * h, axis=-1, keepdims=True) - m * m
    y = (h - m) * lax.rsqrt(v + _EPS) * gn_ref[...] + bbn_ref[...]
    out_ref[0] = y

    @pl.when(j == 0)
    def _():
        agg_ref[...] = jnp.zeros_like(agg_ref)

    agg_ref[0] += jnp.sum(y, axis=0, keepdims=True)


def _global_body(D, g_ref, aggv_ref, agge_ref, vn_ref, ve_ref, Wg_ref,
                 bg_ref, gg_ref, bbg_ref, out_ref):
    g = g_ref[...]
    vn = vn_ref[...]
    ve = ve_ref[...]
    aggv = aggv_ref[...] / jnp.where(vn == 0.0, 1.0, vn)
    agge = agge_ref[...] / jnp.where(ve == 0.0, 1.0, ve)
    acc = jnp.dot(g, Wg_ref[0:D], preferred_element_type=jnp.float32)
    acc += jnp.dot(aggv, Wg_ref[D:2 * D], preferred_element_type=jnp.float32)
    acc += jnp.dot(agge, Wg_ref[2 * D:3 * D], preferred_element_type=jnp.float32)
    acc += bg_ref[...]
    h = jnp.maximum(acc, 0.0) + g
    out_ref[...] = h * (1.0 / jnp.sqrt(1.0 + _EPS)) * gg_ref[...] + bbg_ref[...]


def kernel(vertex_feat, edges_feat, global_feat, edges_idx,
           connected_edges_idx, valid_lens, valid_nodes, valid_edges,
           W_e, b_e, g_e, beta_e, W_n, b_n, g_n, beta_n, W_g, b_g, g_g, beta_g):
    B, N, D = vertex_feat.shape
    E = edges_feat.shape[1]
    K = connected_edges_idx.shape[2]
    BE = 1024
    BN = 1024
    NH = 2
    B2 = B // NH

    gf3 = global_feat[:, None, :]
    b_e2 = b_e.reshape(1, D)
    g_e2 = g_e.reshape(1, D)
    beta_e2 = beta_e.reshape(1, D)
    b_n2 = b_n.reshape(1, D)
    g_n2 = g_n.reshape(1, D)
    beta_n2 = beta_n.reshape(1, D)
    We_bf = W_e.astype(jnp.bfloat16)
    Wn_bf = W_n.astype(jnp.bfloat16)
    vl3 = valid_lens.astype(jnp.float32)[:, :, None]
    offs_n2 = (jnp.arange(B2, dtype=jnp.int32) * N)[:, None]
    offs_e2 = (jnp.arange(B2, dtype=jnp.int32) * E)[:, None, None]

    vtab = vertex_feat.reshape(B * N, D)

    def edge_stage(h):
        ei = edges_idx[h * B2:(h + 1) * B2]
        src = (ei[:, :, 0] + offs_n2 + h * B2 * N).reshape(-1)
        dst = (ei[:, :, 1] + offs_n2 + h * B2 * N).reshape(-1)
        vsr = _sc_gather_rows(vtab, jnp.concatenate([src, dst]))
        vsr = vsr.reshape(2, B2, E, D)
        return pl.pallas_call(
            functools.partial(_edge_body, D),
            grid=(B2, E // BE),
            in_specs=[
                pl.BlockSpec((1, BE, D), lambda b, j: (b + h * B2, j, 0)),
                pl.BlockSpec((2, 1, BE, D), lambda b, j: (0, b, j, 0)),
                pl.BlockSpec((1, 1, D), lambda b, j: (b + h * B2, 0, 0)),
                pl.BlockSpec((4 * D, D), lambda b, j: (0, 0)),
                pl.BlockSpec((1, D), lambda b, j: (0, 0)),
                pl.BlockSpec((1, D), lambda b, j: (0, 0)),
                pl.BlockSpec((1, D), lambda b, j: (0, 0)),
            ],
            out_specs=[
                pl.BlockSpec((1, BE, D), lambda b, j: (b, j, 0)),
                pl.BlockSpec((1, 1, D), lambda b, j: (b, 0, 0)),
            ],
            out_shape=[
                jax.ShapeDtypeStruct((B2, E, D), jnp.float32),
                jax.ShapeDtypeStruct((B2, 1, D), jnp.float32),
            ],
        )(edges_feat, vsr, gf3, We_bf, b_e2, g_e2, beta_e2)

    def node_stage(h, edges_new_h, vertex_new_prev):
        cidx = (connected_edges_idx[h * B2:(h + 1) * B2] + offs_e2).reshape(-1)
        ce = _sc_gather_sum(edges_new_h.reshape(B2 * E, D), cidx, K)
        ce = ce.reshape(B2, N, D)
        in_specs = [
            pl.BlockSpec((1, BN, D), lambda b, j: (b + h * B2, j, 0)),
            pl.BlockSpec((1, BN, D), lambda b, j: (b, j, 0)),
            pl.BlockSpec((1, BN, 1), lambda b, j: (b + h * B2, j, 0)),
            pl.BlockSpec((1, 1, D), lambda b, j: (b + h * B2, 0, 0)),
            pl.BlockSpec((3 * D, D), lambda b, j: (0, 0)),
            pl.BlockSpec((1, D), lambda b, j: (0, 0)),
            pl.BlockSpec((1, D), lambda b, j: (0, 0)),
            pl.BlockSpec((1, D), lambda b, j: (0, 0)),
        ]
        args = [vertex_feat, ce, vl3, gf3, Wn_bf, b_n2, g_n2, beta_n2]
        aliases = {}
        if h > 0:
            in_specs.append(pl.BlockSpec(memory_space=pl.ANY))
            args.append(vertex_new_prev)
            aliases = {8: 0}
        return pl.pallas_call(
            functools.partial(_node_body, D),
            grid=(B2, N // BN),
            in_specs=in_specs,
            out_specs=[
                pl.BlockSpec((1, BN, D), lambda b, j: (b + h * B2, j, 0)),
                pl.BlockSpec((1, 1, D), lambda b, j: (b, 0, 0)),
            ],
            out_shape=[
                jax.ShapeDtypeStruct((B, N, D), jnp.float32),
                jax.ShapeDtypeStruct((B2, 1, D), jnp.float32),
            ],
            input_output_aliases=aliases,
        )(*args)

    en_h, ae_h, av_h = [], [], []
    for h in range(NH):
        en, ae = edge_stage(h)
        en_h.append(en)
        ae_h.append(ae)
    vertex_new = None
    for h in range(NH):
        vertex_new, av = node_stage(h, en_h[h], vertex_new)
        av_h.append(av)
    edges_new = jnp.concatenate(en_h, axis=0)
    agg_e = jnp.concatenate(ae_h, axis=0)
    agg_v = jnp.concatenate(av_h, axis=0)

    vn2 = valid_nodes.astype(jnp.float32)[:, None]
    ve2 = valid_edges.astype(jnp.float32)[:, None]
    b_g2 = b_g.reshape(1, D)
    g_g2 = g_g.reshape(1, D)
    beta_g2 = beta_g.reshape(1, D)
    global_new = pl.pallas_call(
        functools.partial(_global_body, D),
        out_shape=jax.ShapeDtypeStruct((B, D), jnp.float32),
    )(global_feat, agg_v.reshape(B, D), agg_e.reshape(B, D), vn2, ve2,
      W_g, b_g2, g_g2, beta_g2)

    return (vertex_new, edges_new, global_new)
